# Initial kernel scaffold; baseline (speedup 1.0000x reference)
import jax
import jax.numpy as jnp
from jax.experimental import pallas as pl


def kernel(x, edge_index, W1, b1, W2, b2):
    raise NotImplementedError("write your pallas kernel here")


# speedup vs baseline: 24.6967x; 24.6967x over previous
import functools

import jax
import jax.numpy as jnp
from jax import lax
from jax.experimental import pallas as pl
from jax.experimental.pallas import tpu as pltpu
from jax.experimental.pallas import tpu_sc as plsc

N = 10000
E = 320000
D_IN = 128
D_HID = 16
D_OUT = 64

NC = 2
NS = 16
NW = NC * NS
EPT = E // NW
B = 80
NB = EPT // B
N_PAD = 10240
RPT = N_PAD // NS

_MESH = dict(core_axis_name="c", subcore_axis_name="s", num_cores=NC,
             num_subcores=NS)

_SC_CACHE = {}

_SC_PARAMS = pltpu.CompilerParams(use_tc_tiling_on_sc=False)


def _make_edge_pass(d):
  @functools.partial(
      pl.kernel,
      out_type=jax.ShapeDtypeStruct((NC, N_PAD, d), jnp.float32),
      mesh=plsc.VectorSubcoreMesh(**_MESH),
      scratch_types=[
          pltpu.VMEM((NB, B), jnp.int32),
          pltpu.VMEM((NB, B), jnp.int32),
          pltpu.VMEM((B, d), jnp.float32),
          pltpu.VMEM_SHARED((N_PAD, d), jnp.float32),
      ],
      compiler_params=_SC_PARAMS,
  )
  def edge_pass(y_hbm, src_hbm, dst_hbm, zero_hbm, out_hbm, srcv, dstv, buf,
                acc):
    cid = lax.axis_index("c")
    sid = lax.axis_index("s")
    wid = sid * NC + cid
    pltpu.sync_copy(src_hbm.at[wid], srcv)
    pltpu.sync_copy(dst_hbm.at[wid], dstv)
    pltpu.sync_copy(zero_hbm.at[pl.ds(sid * RPT, RPT)],
                    acc.at[pl.ds(sid * RPT, RPT)])
    plsc.subcore_barrier()

    def step(j, carry):
      pltpu.sync_copy(y_hbm.at[srcv.at[j]], buf)
      pltpu.sync_copy(buf, acc.at[dstv.at[j]], add=True)
      return carry

    lax.fori_loop(0, NB, step, 0, unroll=False)
    plsc.subcore_barrier()
    pltpu.sync_copy(acc.at[pl.ds(sid * RPT, RPT)],
                    out_hbm.at[cid, pl.ds(sid * RPT, RPT)])

  return edge_pass


def _make_deg_pass():
  @functools.partial(
      pl.kernel,
      out_type=jax.ShapeDtypeStruct((NC, N_PAD, D_HID), jnp.float32),
      mesh=plsc.VectorSubcoreMesh(**_MESH),
      scratch_types=[
          pltpu.VMEM((NB, B), jnp.int32),
          pltpu.VMEM((B, D_HID), jnp.float32),
          pltpu.VMEM_SHARED((N_PAD, D_HID), jnp.float32),
      ],
      compiler_params=_SC_PARAMS,
  )
  def deg_pass(dst_hbm, ones_hbm, zero_hbm, out_hbm, dstv, buf, acc):
    cid = lax.axis_index("c")
    sid = lax.axis_index("s")
    wid = sid * NC + cid
    pltpu.sync_copy(dst_hbm.at[wid], dstv)
    pltpu.sync_copy(ones_hbm, buf)
    pltpu.sync_copy(zero_hbm.at[pl.ds(sid * RPT, RPT)],
                    acc.at[pl.ds(sid * RPT, RPT)])
    plsc.subcore_barrier()

    def step(j, carry):
      pltpu.sync_copy(buf, acc.at[dstv.at[j]], add=True)
      return carry

    lax.fori_loop(0, NB, step, 0, unroll=False)
    plsc.subcore_barrier()
    pltpu.sync_copy(acc.at[pl.ds(sid * RPT, RPT)],
                    out_hbm.at[cid, pl.ds(sid * RPT, RPT)])

  return deg_pass


def _sc_kernels():
  if not _SC_CACHE:
    _SC_CACHE["edge16"] = _make_edge_pass(D_HID)
    _SC_CACHE["edge64"] = _make_edge_pass(D_OUT)
    _SC_CACHE["deg"] = _make_deg_pass()
  return _SC_CACHE["deg"], _SC_CACHE["edge16"], _SC_CACHE["edge64"]


_BM = 2000


def _tc1_body(d0, d1, x, w1, yo, dio):
  deg = d0[:, 0:1] + d1[:, 0:1] + 1.0
  dinv = lax.rsqrt(deg)
  dio[...] = dinv
  yo[...] = jnp.dot(x[...], w1[...], preferred_element_type=jnp.float32) * dinv


def _tc2_body(p0, p1, y, dinv, w2, b1, zo):
  di = dinv[...]
  h = jnp.maximum((p0[...] + p1[...] + y[...]) * di + b1[...], 0.0)
  zo[...] = jnp.dot(h, w2[...], preferred_element_type=jnp.float32) * di


def _tc3_body(p0, p1, z, dinv, b2, o):
  a = (p0[...] + p1[...] + z[...]) * dinv[...] + b2[...]
  m = jnp.max(a, axis=1, keepdims=True)
  ex = jnp.exp(a - m)
  o[...] = a - (jnp.log(jnp.sum(ex, axis=1, keepdims=True)) + m)


def _row_spec(d):
  return pl.BlockSpec((_BM, d), lambda m: (m, 0))


def _full_spec(r, d):
  return pl.BlockSpec((r, d), lambda m: (0, 0))


_GRID = N // _BM

_tc1 = pl.pallas_call(
    _tc1_body,
    grid=(_GRID,),
    in_specs=[_row_spec(D_HID), _row_spec(D_HID), _row_spec(D_IN),
              _full_spec(D_IN, D_HID)],
    out_specs=[_row_spec(D_HID), _row_spec(1)],
    out_shape=[jax.ShapeDtypeStruct((N, D_HID), jnp.float32),
               jax.ShapeDtypeStruct((N, 1), jnp.float32)],
)

_tc2 = pl.pallas_call(
    _tc2_body,
    grid=(_GRID,),
    in_specs=[_row_spec(D_HID), _row_spec(D_HID), _row_spec(D_HID),
              _row_spec(1), _full_spec(D_HID, D_OUT), _full_spec(1, D_HID)],
    out_specs=_row_spec(D_OUT),
    out_shape=jax.ShapeDtypeStruct((N, D_OUT), jnp.float32),
)

_tc3 = pl.pallas_call(
    _tc3_body,
    grid=(_GRID,),
    in_specs=[_row_spec(D_OUT), _row_spec(D_OUT), _row_spec(D_OUT),
              _row_spec(1), _full_spec(1, D_OUT)],
    out_specs=_row_spec(D_OUT),
    out_shape=jax.ShapeDtypeStruct((N, D_OUT), jnp.float32),
)


def kernel(x, edge_index, W1, b1, W2, b2):
  ei = edge_index.astype(jnp.int32)
  src3 = ei[0].reshape(NW, NB, B)
  dst3 = ei[1].reshape(NW, NB, B)
  zeros16 = jnp.zeros((N_PAD, D_HID), jnp.float32)
  zeros64 = jnp.zeros((N_PAD, D_OUT), jnp.float32)
  onesb = jnp.ones((B, D_HID), jnp.float32)

  _deg, _edge16, _edge64 = _sc_kernels()
  degp = _deg(dst3, onesb, zeros16)
  y1, dinv = _tc1(degp[0, :N], degp[1, :N], x, W1)
  p1 = _edge16(y1, src3, dst3, zeros16)
  z = _tc2(p1[0, :N], p1[1, :N], y1, dinv, W2, b1.reshape(1, D_HID))
  p2 = _edge64(z, src3, dst3, zeros64)
  return _tc3(p2[0, :N], p2[1, :N], z, dinv, b2.reshape(1, D_OUT))


# speedup vs baseline: 34.3352x; 1.3903x over previous
import functools

import jax
import jax.numpy as jnp
from jax import lax
from jax.experimental import pallas as pl
from jax.experimental.pallas import tpu as pltpu
from jax.experimental.pallas import tpu_sc as plsc

N = 10000
E = 320000
D_IN = 128
D_HID = 16
D_OUT = 64

NC = 2
NS = 16
NW = NC * NS
EPT = E // NW
B = 80
NB = EPT // B
N_PAD = 10240
RPT = N_PAD // NS

_MESH = dict(core_axis_name="c", subcore_axis_name="s", num_cores=NC,
             num_subcores=NS)

_SC_CACHE = {}

_SC_PARAMS = pltpu.CompilerParams(use_tc_tiling_on_sc=False)


def _make_edge_pass(d):
  @functools.partial(
      pl.kernel,
      out_type=jax.ShapeDtypeStruct((NC, N_PAD, d), jnp.float32),
      mesh=plsc.VectorSubcoreMesh(**_MESH),
      scratch_types=[
          pltpu.VMEM((NB, B), jnp.int32),
          pltpu.VMEM((NB, B), jnp.int32),
          pltpu.VMEM((B, d), jnp.float32),
          pltpu.VMEM((B, d), jnp.float32),
          pltpu.SemaphoreType.DMA,
          pltpu.SemaphoreType.DMA,
          pltpu.VMEM_SHARED((N_PAD, d), jnp.float32),
      ],
      compiler_params=_SC_PARAMS,
  )
  def edge_pass(y_hbm, src_hbm, dst_hbm, zero_hbm, out_hbm, srcv, dstv, buf0,
                buf1, sem0, sem1, acc):
    cid = lax.axis_index("c")
    sid = lax.axis_index("s")
    wid = sid * NC + cid
    pltpu.sync_copy(src_hbm.at[wid], srcv)
    pltpu.sync_copy(dst_hbm.at[wid], dstv)
    pltpu.sync_copy(zero_hbm.at[pl.ds(sid * RPT, RPT)],
                    acc.at[pl.ds(sid * RPT, RPT)])
    plsc.subcore_barrier()

    bufs = (buf0, buf1)
    sems = (sem0, sem1)

    def gather(j, b):
      pltpu.async_copy(y_hbm.at[srcv.at[j]], bufs[b], sems[b])

    def gwait(j, b):
      pltpu.make_async_copy(y_hbm.at[srcv.at[j]], bufs[b], sems[b]).wait()

    def scat(j, b):
      pltpu.sync_copy(bufs[b], acc.at[dstv.at[j]], add=True)

    gather(0, 0)

    def step(i, carry):
      j0 = 2 * i
      gather(j0 + 1, 1)
      gwait(j0, 0)
      scat(j0, 0)
      gather(j0 + 2, 0)
      gwait(j0 + 1, 1)
      scat(j0 + 1, 1)
      return carry

    lax.fori_loop(0, (NB - 1) // 2, step, 0, unroll=False)
    gwait(NB - 1, 0)
    scat(NB - 1, 0)
    plsc.subcore_barrier()
    pltpu.sync_copy(acc.at[pl.ds(sid * RPT, RPT)],
                    out_hbm.at[cid, pl.ds(sid * RPT, RPT)])

  return edge_pass


def _make_deg_pass():
  @functools.partial(
      pl.kernel,
      out_type=jax.ShapeDtypeStruct((NC, N_PAD, D_HID), jnp.float32),
      mesh=plsc.VectorSubcoreMesh(**_MESH),
      scratch_types=[
          pltpu.VMEM((NB, B), jnp.int32),
          pltpu.VMEM((B, D_HID), jnp.float32),
          pltpu.VMEM_SHARED((N_PAD, D_HID), jnp.float32),
      ],
      compiler_params=_SC_PARAMS,
  )
  def deg_pass(dst_hbm, ones_hbm, zero_hbm, out_hbm, dstv, buf, acc):
    cid = lax.axis_index("c")
    sid = lax.axis_index("s")
    wid = sid * NC + cid
    pltpu.sync_copy(dst_hbm.at[wid], dstv)
    pltpu.sync_copy(ones_hbm, buf)
    pltpu.sync_copy(zero_hbm.at[pl.ds(sid * RPT, RPT)],
                    acc.at[pl.ds(sid * RPT, RPT)])
    plsc.subcore_barrier()

    def step(j, carry):
      pltpu.sync_copy(buf, acc.at[dstv.at[j]], add=True)
      return carry

    lax.fori_loop(0, NB, step, 0, unroll=False)
    plsc.subcore_barrier()
    pltpu.sync_copy(acc.at[pl.ds(sid * RPT, RPT)],
                    out_hbm.at[cid, pl.ds(sid * RPT, RPT)])

  return deg_pass


def _sc_kernels():
  if not _SC_CACHE:
    _SC_CACHE["edge16"] = _make_edge_pass(D_HID)
    _SC_CACHE["edge64"] = _make_edge_pass(D_OUT)
    _SC_CACHE["deg"] = _make_deg_pass()
  return _SC_CACHE["deg"], _SC_CACHE["edge16"], _SC_CACHE["edge64"]


_BM = 2000


def _tc1_body(d0, d1, x, w1, yo, dio):
  deg = d0[:, 0:1] + d1[:, 0:1] + 1.0
  dinv = lax.rsqrt(deg)
  dio[...] = dinv
  yo[...] = jnp.dot(x[...], w1[...], preferred_element_type=jnp.float32) * dinv


def _tc2_body(p0, p1, y, dinv, w2, b1, zo):
  di = dinv[...]
  h = jnp.maximum((p0[...] + p1[...] + y[...]) * di + b1[...], 0.0)
  zo[...] = jnp.dot(h, w2[...], preferred_element_type=jnp.float32) * di


def _tc3_body(p0, p1, z, dinv, b2, o):
  a = (p0[...] + p1[...] + z[...]) * dinv[...] + b2[...]
  m = jnp.max(a, axis=1, keepdims=True)
  ex = jnp.exp(a - m)
  o[...] = a - (jnp.log(jnp.sum(ex, axis=1, keepdims=True)) + m)


def _row_spec(d):
  return pl.BlockSpec((_BM, d), lambda m: (m, 0))


def _full_spec(r, d):
  return pl.BlockSpec((r, d), lambda m: (0, 0))


_GRID = N // _BM

_tc1 = pl.pallas_call(
    _tc1_body,
    grid=(_GRID,),
    in_specs=[_row_spec(D_HID), _row_spec(D_HID), _row_spec(D_IN),
              _full_spec(D_IN, D_HID)],
    out_specs=[_row_spec(D_HID), _row_spec(1)],
    out_shape=[jax.ShapeDtypeStruct((N, D_HID), jnp.float32),
               jax.ShapeDtypeStruct((N, 1), jnp.float32)],
)

_tc2 = pl.pallas_call(
    _tc2_body,
    grid=(_GRID,),
    in_specs=[_row_spec(D_HID), _row_spec(D_HID), _row_spec(D_HID),
              _row_spec(1), _full_spec(D_HID, D_OUT), _full_spec(1, D_HID)],
    out_specs=_row_spec(D_OUT),
    out_shape=jax.ShapeDtypeStruct((N, D_OUT), jnp.float32),
)

_tc3 = pl.pallas_call(
    _tc3_body,
    grid=(_GRID,),
    in_specs=[_row_spec(D_OUT), _row_spec(D_OUT), _row_spec(D_OUT),
              _row_spec(1), _full_spec(1, D_OUT)],
    out_specs=_row_spec(D_OUT),
    out_shape=jax.ShapeDtypeStruct((N, D_OUT), jnp.float32),
)


def kernel(x, edge_index, W1, b1, W2, b2):
  ei = edge_index.astype(jnp.int32)
  src3 = ei[0].reshape(NW, NB, B)
  dst3 = ei[1].reshape(NW, NB, B)
  zeros16 = jnp.zeros((N_PAD, D_HID), jnp.float32)
  zeros64 = jnp.zeros((N_PAD, D_OUT), jnp.float32)
  onesb = jnp.ones((B, D_HID), jnp.float32)

  _deg, _edge16, _edge64 = _sc_kernels()
  degp = _deg(dst3, onesb, zeros16)
  y1, dinv = _tc1(degp[0, :N], degp[1, :N], x, W1)
  p1 = _edge16(y1, src3, dst3, zeros16)
  z = _tc2(p1[0, :N], p1[1, :N], y1, dinv, W2, b1.reshape(1, D_HID))
  p2 = _edge64(z, src3, dst3, zeros64)
  return _tc3(p2[0, :N], p2[1, :N], z, dinv, b2.reshape(1, D_OUT))


# speedup vs baseline: 35.8257x; 1.0434x over previous
import functools

import jax
import jax.numpy as jnp
from jax import lax
from jax.experimental import pallas as pl
from jax.experimental.pallas import tpu as pltpu
from jax.experimental.pallas import tpu_sc as plsc

N = 10000
E = 320000
D_IN = 128
D_HID = 16
D_OUT = 64

NC = 2
NS = 16
NW = NC * NS
EPT = E // NW
B = 80
NB = EPT // B
N_PAD = 10240
RPT = N_PAD // NS

_MESH = dict(core_axis_name="c", subcore_axis_name="s", num_cores=NC,
             num_subcores=NS)

_SC_CACHE = {}

_SC_PARAMS = pltpu.CompilerParams(use_tc_tiling_on_sc=False)


def _make_edge_pass(d):
  @functools.partial(
      pl.kernel,
      out_type=jax.ShapeDtypeStruct((NC, N_PAD, d), jnp.float32),
      mesh=plsc.VectorSubcoreMesh(**_MESH),
      scratch_types=[
          pltpu.VMEM((NB, B), jnp.int32),
          pltpu.VMEM((NB, B), jnp.int32),
          pltpu.VMEM((B, d), jnp.float32),
          pltpu.VMEM((B, d), jnp.float32),
          pltpu.SemaphoreType.DMA,
          pltpu.SemaphoreType.DMA,
          pltpu.VMEM_SHARED((N_PAD, d), jnp.float32),
      ],
      compiler_params=_SC_PARAMS,
  )
  def edge_pass(y_hbm, src_hbm, dst_hbm, out_hbm, srcv, dstv, buf0,
                buf1, sem0, sem1, acc):
    cid = lax.axis_index("c")
    sid = lax.axis_index("s")
    wid = sid * NC + cid
    pltpu.sync_copy(src_hbm.at[wid], srcv)
    pltpu.sync_copy(dst_hbm.at[wid], dstv)

    def zrow(i, c):
      for cc in range(d // 16):
        buf0[i, pl.ds(cc * 16, 16)] = jnp.zeros((16,), jnp.float32)
      return c

    lax.fori_loop(0, B, zrow, 0, unroll=False)
    for r in range(RPT // B):
      pltpu.sync_copy(buf0, acc.at[pl.ds(sid * RPT + r * B, B)])
    plsc.subcore_barrier()

    bufs = (buf0, buf1)
    sems = (sem0, sem1)

    def gather(j, b):
      pltpu.async_copy(y_hbm.at[srcv.at[j]], bufs[b], sems[b])

    def gwait(j, b):
      pltpu.make_async_copy(y_hbm.at[srcv.at[j]], bufs[b], sems[b]).wait()

    def scat(j, b):
      pltpu.sync_copy(bufs[b], acc.at[dstv.at[j]], add=True)

    gather(0, 0)

    def step(i, carry):
      j0 = 2 * i
      gather(j0 + 1, 1)
      gwait(j0, 0)
      scat(j0, 0)
      gather(j0 + 2, 0)
      gwait(j0 + 1, 1)
      scat(j0 + 1, 1)
      return carry

    lax.fori_loop(0, (NB - 1) // 2, step, 0, unroll=False)
    gwait(NB - 1, 0)
    scat(NB - 1, 0)
    plsc.subcore_barrier()
    pltpu.sync_copy(acc.at[pl.ds(sid * RPT, RPT)],
                    out_hbm.at[cid, pl.ds(sid * RPT, RPT)])

  return edge_pass


def _make_deg_pass():
  @functools.partial(
      pl.kernel,
      out_type=jax.ShapeDtypeStruct((NC, N_PAD, D_HID), jnp.float32),
      mesh=plsc.VectorSubcoreMesh(**_MESH),
      scratch_types=[
          pltpu.VMEM((NB, B), jnp.int32),
          pltpu.VMEM((B, D_HID), jnp.float32),
          pltpu.SemaphoreType.DMA,
          pltpu.VMEM_SHARED((N_PAD, D_HID), jnp.float32),
      ],
      compiler_params=_SC_PARAMS,
  )
  def deg_pass(dst_hbm, out_hbm, dstv, buf, sem, acc):
    cid = lax.axis_index("c")
    sid = lax.axis_index("s")
    wid = sid * NC + cid
    pltpu.sync_copy(dst_hbm.at[wid], dstv)

    def fill(val):
      def frow(i, c):
        buf[i, :] = jnp.full((16,), val, jnp.float32)
        return c
      lax.fori_loop(0, B, frow, 0, unroll=False)

    fill(0.0)
    for r in range(RPT // B):
      pltpu.sync_copy(buf, acc.at[pl.ds(sid * RPT + r * B, B)])
    fill(1.0)
    plsc.subcore_barrier()

    def fire(j, c):
      pltpu.async_copy(buf, acc.at[dstv.at[j]], sem, add=True)
      return c

    def drain(j, c):
      pltpu.make_async_copy(buf, acc.at[dstv.at[j]], sem).wait()
      return c

    lax.fori_loop(0, NB, fire, 0, unroll=False)
    lax.fori_loop(0, NB, drain, 0, unroll=False)
    plsc.subcore_barrier()
    pltpu.sync_copy(acc.at[pl.ds(sid * RPT, RPT)],
                    out_hbm.at[cid, pl.ds(sid * RPT, RPT)])

  return deg_pass


def _sc_kernels():
  if not _SC_CACHE:
    _SC_CACHE["edge16"] = _make_edge_pass(D_HID)
    _SC_CACHE["edge64"] = _make_edge_pass(D_OUT)
    _SC_CACHE["deg"] = _make_deg_pass()
  return _SC_CACHE["deg"], _SC_CACHE["edge16"], _SC_CACHE["edge64"]


_BM = 2000


def _tc0_body(x, w1, ho):
  ho[...] = jnp.dot(x[...], w1[...], preferred_element_type=jnp.float32)


def _tc1_body(d0, d1, h, yo, dio):
  deg = d0[:, 0:1] + d1[:, 0:1] + 1.0
  dinv = lax.rsqrt(deg)
  dio[...] = dinv
  yo[...] = h[...] * dinv


def _tc2_body(p0, p1, y, dinv, w2, b1, zo):
  di = dinv[...]
  h = jnp.maximum((p0[...] + p1[...] + y[...]) * di + b1[...], 0.0)
  zo[...] = jnp.dot(h, w2[...], preferred_element_type=jnp.float32) * di


def _tc3_body(p0, p1, z, dinv, b2, o):
  a = (p0[...] + p1[...] + z[...]) * dinv[...] + b2[...]
  m = jnp.max(a, axis=1, keepdims=True)
  ex = jnp.exp(a - m)
  o[...] = a - (jnp.log(jnp.sum(ex, axis=1, keepdims=True)) + m)


def _row_spec(d):
  return pl.BlockSpec((_BM, d), lambda m: (m, 0))


def _full_spec(r, d):
  return pl.BlockSpec((r, d), lambda m: (0, 0))


_GRID = N // _BM

_tc0 = pl.pallas_call(
    _tc0_body,
    grid=(_GRID,),
    in_specs=[_row_spec(D_IN), _full_spec(D_IN, D_HID)],
    out_specs=_row_spec(D_HID),
    out_shape=jax.ShapeDtypeStruct((N, D_HID), jnp.float32),
)

_tc1 = pl.pallas_call(
    _tc1_body,
    grid=(_GRID,),
    in_specs=[_row_spec(D_HID), _row_spec(D_HID), _row_spec(D_HID)],
    out_specs=[_row_spec(D_HID), _row_spec(1)],
    out_shape=[jax.ShapeDtypeStruct((N, D_HID), jnp.float32),
               jax.ShapeDtypeStruct((N, 1), jnp.float32)],
)

_tc2 = pl.pallas_call(
    _tc2_body,
    grid=(_GRID,),
    in_specs=[_row_spec(D_HID), _row_spec(D_HID), _row_spec(D_HID),
              _row_spec(1), _full_spec(D_HID, D_OUT), _full_spec(1, D_HID)],
    out_specs=_row_spec(D_OUT),
    out_shape=jax.ShapeDtypeStruct((N, D_OUT), jnp.float32),
)

_tc3 = pl.pallas_call(
    _tc3_body,
    grid=(_GRID,),
    in_specs=[_row_spec(D_OUT), _row_spec(D_OUT), _row_spec(D_OUT),
              _row_spec(1), _full_spec(1, D_OUT)],
    out_specs=_row_spec(D_OUT),
    out_shape=jax.ShapeDtypeStruct((N, D_OUT), jnp.float32),
)


def kernel(x, edge_index, W1, b1, W2, b2):
  ei = edge_index.astype(jnp.int32)
  src3 = ei[0].reshape(NW, NB, B)
  dst3 = ei[1].reshape(NW, NB, B)
  _deg, _edge16, _edge64 = _sc_kernels()
  degp = _deg(dst3)
  h1 = _tc0(x, W1)
  y1, dinv = _tc1(degp[0, :N], degp[1, :N], h1)
  p1 = _edge16(y1, src3, dst3)
  z = _tc2(p1[0, :N], p1[1, :N], y1, dinv, W2, b1.reshape(1, D_HID))
  p2 = _edge64(z, src3, dst3)
  return _tc3(p2[0, :N], p2[1, :N], z, dinv, b2.reshape(1, D_OUT))
